# Initial kernel scaffold; baseline (speedup 1.0000x reference)
import jax
import jax.numpy as jnp
from jax.experimental import pallas as pl


def kernel(product_id, product_embed_weight):
    raise NotImplementedError("write your pallas kernel here")


# speedup vs baseline: 3.5521x; 3.5521x over previous
import functools

import jax
import jax.numpy as jnp
from jax import lax
from jax.experimental import pallas as pl
from jax.experimental.pallas import tpu as pltpu
from jax.experimental.pallas import tpu_sc as plsc

NC, NS = 2, 16
NW = NC * NS
B = 16384 * 50
D = 64
B_PER_W = B // NW
CHUNK = 800
NCHUNK = B_PER_W // CHUNK

_mesh = plsc.VectorSubcoreMesh(core_axis_name="c", subcore_axis_name="s")


@functools.partial(
    pl.kernel,
    out_type=jax.ShapeDtypeStruct((B, D), jnp.float32),
    mesh=_mesh,
    scratch_types=[
        pltpu.VMEM((CHUNK,), jnp.int32),
        pltpu.VMEM((CHUNK, D), jnp.float32),
        pltpu.SemaphoreType.DMA,
    ],
    compiler_params=pltpu.CompilerParams(use_tc_tiling_on_sc=False),
)
def _embed(idx_hbm, table_hbm, out_hbm, idx_v, rows_v, sem):
    wid = lax.axis_index("s") * NC + lax.axis_index("c")
    base = wid * B_PER_W

    @pl.loop(0, NCHUNK)
    def _chunk(c):
        off = base + c * CHUNK
        pltpu.sync_copy(idx_hbm.at[pl.ds(off, CHUNK)], idx_v)
        pltpu.async_copy(table_hbm.at[idx_v], rows_v, sem).wait()
        pltpu.sync_copy(rows_v, out_hbm.at[pl.ds(off, CHUNK)])


def kernel(product_id, product_embed_weight):
    flat = product_id.reshape(-1)
    out = _embed(flat, product_embed_weight)
    return out.reshape(product_id.shape + (D,))


# speedup vs baseline: 3.5569x; 1.0014x over previous
import functools

import jax
import jax.numpy as jnp
from jax import lax
from jax.experimental import pallas as pl
from jax.experimental.pallas import tpu as pltpu
from jax.experimental.pallas import tpu_sc as plsc

NC, NS = 2, 16
NW = NC * NS
B = 16384 * 50
D = 64
B_PER_W = B // NW
CHUNK = 400
NCHUNK = B_PER_W // CHUNK
NBUF = 4

_mesh = plsc.VectorSubcoreMesh(core_axis_name="c", subcore_axis_name="s")


@functools.partial(
    pl.kernel,
    out_type=jax.ShapeDtypeStruct((B, D), jnp.float32),
    mesh=_mesh,
    scratch_types=[
        pltpu.VMEM((B_PER_W,), jnp.int32),
        [pltpu.VMEM((CHUNK, D), jnp.float32) for _ in range(NBUF)],
        [pltpu.SemaphoreType.DMA for _ in range(NBUF)],
        [pltpu.SemaphoreType.DMA for _ in range(NBUF)],
    ],
    compiler_params=pltpu.CompilerParams(use_tc_tiling_on_sc=False),
)
def _embed(idx_hbm, table_hbm, out_hbm, idx_all, rows, gsem, ssem):
    wid = lax.axis_index("s") * NC + lax.axis_index("c")
    base = wid * B_PER_W

    pltpu.sync_copy(idx_hbm.at[pl.ds(base, B_PER_W)], idx_all)

    def gather(c):
        b = c % NBUF
        return pltpu.async_copy(
            table_hbm.at[idx_all.at[pl.ds(c * CHUNK, CHUNK)]], rows[b], gsem[b]
        )

    def store(c):
        b = c % NBUF
        return pltpu.async_copy(
            rows[b], out_hbm.at[pl.ds(base + c * CHUNK, CHUNK)], ssem[b]
        )

    gathers = [None] * NCHUNK
    stores = [None] * NCHUNK
    for c in range(NBUF):
        gathers[c] = gather(c)
    for c in range(NCHUNK):
        gathers[c].wait()
        stores[c] = store(c)
        nxt = c + NBUF
        if nxt < NCHUNK:
            stores[c].wait()
            gathers[nxt] = gather(nxt)
    for c in range(NCHUNK - NBUF, NCHUNK):
        if stores[c] is not None:
            stores[c].wait()


def kernel(product_id, product_embed_weight):
    flat = product_id.reshape(-1)
    out = _embed(flat, product_embed_weight)
    return out.reshape(product_id.shape + (D,))


# speedup vs baseline: 3.5634x; 1.0018x over previous
import functools

import jax
import jax.numpy as jnp
from jax import lax
from jax.experimental import pallas as pl
from jax.experimental.pallas import tpu as pltpu
from jax.experimental.pallas import tpu_sc as plsc

NC, NS = 2, 16
NW = NC * NS
R = 16384
S = 50
D = 64
R_PER_W = R // NW
GROUP = 4
NSLOT = 4
NGRP = R_PER_W // GROUP

_mesh = plsc.VectorSubcoreMesh(core_axis_name="c", subcore_axis_name="s")


@functools.partial(
    pl.kernel,
    out_type=jax.ShapeDtypeStruct((R, S, D), jnp.float32),
    mesh=_mesh,
    scratch_types=[
        pltpu.VMEM((R_PER_W, S), jnp.int32),
        pltpu.VMEM((NSLOT * GROUP, S, D), jnp.float32),
        [pltpu.SemaphoreType.DMA for _ in range(NSLOT)],
        [pltpu.SemaphoreType.DMA for _ in range(NSLOT)],
    ],
    compiler_params=pltpu.CompilerParams(use_tc_tiling_on_sc=False),
)
def _embed(idx_hbm, table_hbm, out_hbm, idx_all, rows, gsem, ssem):
    wid = lax.axis_index("s") * NC + lax.axis_index("c")
    base = wid * R_PER_W

    pltpu.sync_copy(idx_hbm.at[pl.ds(base, R_PER_W)], idx_all)

    def fire_gathers(g, slot):
        for k in range(GROUP):
            pltpu.async_copy(
                table_hbm.at[idx_all.at[g * GROUP + k]],
                rows.at[slot * GROUP + k],
                gsem[slot],
            )

    def wait_gathers(g, slot):
        pltpu.make_async_copy(
            out_hbm.at[pl.ds(base + g * GROUP, GROUP)],
            rows.at[pl.ds(slot * GROUP, GROUP)],
            gsem[slot],
        ).wait()

    def fire_store(g, slot):
        return pltpu.async_copy(
            rows.at[pl.ds(slot * GROUP, GROUP)],
            out_hbm.at[pl.ds(base + g * GROUP, GROUP)],
            ssem[slot],
        )

    def wait_store(g, slot):
        pltpu.make_async_copy(
            rows.at[pl.ds(slot * GROUP, GROUP)],
            out_hbm.at[pl.ds(base + g * GROUP, GROUP)],
            ssem[slot],
        ).wait()

    for g in range(NSLOT):
        fire_gathers(g, g)

    @pl.loop(0, NGRP - NSLOT, step=NSLOT)
    def _steady(g0):
        for j in range(NSLOT):
            g = g0 + j
            wait_gathers(g, j)
            fire_store(g, j)
            wait_store(g, j)
            fire_gathers(g + NSLOT, j)

    for j in range(NSLOT):
        g = NGRP - NSLOT + j
        wait_gathers(g, j)
        fire_store(g, j)
        wait_store(g, j)


def kernel(product_id, product_embed_weight):
    return _embed(product_id, product_embed_weight)


# speedup vs baseline: 3.5904x; 1.0076x over previous
import functools

import jax
import jax.numpy as jnp
from jax import lax
from jax.experimental import pallas as pl
from jax.experimental.pallas import tpu as pltpu
from jax.experimental.pallas import tpu_sc as plsc

NC, NS = 2, 16
NW = NC * NS
R = 16384
S = 50
D = 64
V = 100
NI = R // NW
L = 16
NB = NI // L

_mesh = plsc.VectorSubcoreMesh(core_axis_name="c", subcore_axis_name="s")


@functools.partial(
    pl.kernel,
    out_type=jax.ShapeDtypeStruct((S, D, R), jnp.float32),
    mesh=_mesh,
    scratch_types=[
        pltpu.VMEM((V * D,), jnp.float32),
        pltpu.VMEM((S, NI), jnp.int32),
        [pltpu.VMEM((D, NI), jnp.float32) for _ in range(2)],
        pltpu.SemaphoreType.DMA,
        [pltpu.SemaphoreType.DMA for _ in range(2)],
    ],
    compiler_params=pltpu.CompilerParams(
        use_tc_tiling_on_sc=True, needs_layout_passes=False
    ),
)
def _embed(idxT_hbm, tab_hbm, out_hbm, tab_v, idx_v, obuf, tsem, ssem):
    wid = lax.axis_index("s") * NC + lax.axis_index("c")
    i0 = wid * NI

    pltpu.async_copy(tab_hbm, tab_v, tsem).wait()
    pltpu.async_copy(idxT_hbm.at[:, pl.ds(i0, NI)], idx_v, tsem).wait()

    @pl.loop(0, S)
    def _scale(s):
        @pl.loop(0, NB)
        def _scale_b(ib):
            sl = pl.ds(ib * L, L)
            idx_v[s, sl] = idx_v[s, sl] * D

    def compute(s, b):
        @pl.loop(0, NB)
        def _gather(ib):
            sl = pl.ds(ib * L, L)
            a = idx_v[s, sl]
            for d in range(D):
                obuf[b][d, sl] = plsc.load_gather(tab_v, [a + d])

    def fire_store(s, b):
        pltpu.async_copy(obuf[b], out_hbm.at[s, :, pl.ds(i0, NI)], ssem[b])

    def wait_store(b):
        pltpu.make_async_copy(
            obuf[b], out_hbm.at[0, :, pl.ds(i0, NI)], ssem[b]
        ).wait()

    compute(0, 0)
    fire_store(0, 0)
    compute(1, 1)
    fire_store(1, 1)

    @pl.loop(2, S, step=2)
    def _main(s):
        wait_store(0)
        compute(s, 0)
        fire_store(s, 0)
        wait_store(1)
        compute(s + 1, 1)
        fire_store(s + 1, 1)

    wait_store(0)
    wait_store(1)


def kernel(product_id, product_embed_weight):
    idxT = product_id.T
    tab = product_embed_weight.reshape(-1)
    out = _embed(idxT, tab)
    return jnp.transpose(out, (2, 0, 1))
